# Initial kernel scaffold; baseline (speedup 1.0000x reference)
import jax
import jax.numpy as jnp
from jax.experimental import pallas as pl


def kernel(embeddings, inputs):
    raise NotImplementedError("write your pallas kernel here")


# speedup vs baseline: 1.4859x; 1.4859x over previous
import jax
import jax.numpy as jnp
from jax import lax
from jax.experimental import pallas as pl
from jax.experimental.pallas import tpu as pltpu
from jax.experimental.pallas import tpu_sc as plsc

PADDING_IDX = 0
D = 32
LANES = 16
NC, NS = 2, 16
NW = NC * NS
CHUNK = 1280
VREGS = CHUNK // LANES


def _zero_pad_rows(idx_v, rows_v, k):
    iv = idx_v[pl.ds(k * LANES, LANES)]
    m = iv == PADDING_IDX

    @pl.when(jnp.any(m))
    def _():
        row_ix = jnp.arange(LANES, dtype=jnp.int32) + k * LANES
        zeros = jnp.zeros((LANES,), dtype=jnp.float32)

        def col(j, carry):
            col_ix = jnp.full((LANES,), j, dtype=jnp.int32)
            plsc.store_scatter(rows_v, [row_ix, col_ix], zeros, mask=m)
            return carry

        lax.fori_loop(0, D, col, 0)


def _mask_fixup(idx_v, rows_v):
    def acc_any(k, m):
        return m | (idx_v[pl.ds(k * LANES, LANES)] == PADDING_IDX)

    m0 = jnp.zeros((LANES,), dtype=jnp.bool_)
    mall = lax.fori_loop(0, VREGS, acc_any, m0)

    @pl.when(jnp.any(mall))
    def _():
        def per_k(k, carry):
            _zero_pad_rows(idx_v, rows_v, k)
            return carry

        lax.fori_loop(0, VREGS, per_k, 0)


def _sc_body(table_hbm, idx_hbm, out_hbm,
             idx_a, idx_b, rows_a, rows_b, sem_a, sem_b):
    wid = lax.axis_index("s") * NC + lax.axis_index("c")
    per_w = idx_hbm.shape[0] // NW
    base = wid * per_w
    n_pairs = per_w // (2 * CHUNK)

    def stage(g, idx_v, rows_v, sem):
        off = base + g * CHUNK
        pltpu.sync_copy(idx_hbm.at[pl.ds(off, CHUNK)], idx_v)
        return pltpu.async_copy(table_hbm.at[idx_v], rows_v, sem)

    def drain(g, idx_v, rows_v, cp):
        cp.wait()
        _mask_fixup(idx_v, rows_v)
        pltpu.sync_copy(rows_v, out_hbm.at[pl.ds(base + g * CHUNK, CHUNK)])

    def pair_body(p, carry):
        g = p * 2
        cpa = stage(g, idx_a, rows_a, sem_a)
        cpb = stage(g + 1, idx_b, rows_b, sem_b)
        drain(g, idx_a, rows_a, cpa)
        drain(g + 1, idx_b, rows_b, cpb)
        return carry

    lax.fori_loop(0, n_pairs, pair_body, 0)


def kernel(embeddings, inputs):
    B, L = inputs.shape
    total = B * L
    idx1d = inputs.reshape(total).astype(jnp.int32)

    mesh = plsc.VectorSubcoreMesh(core_axis_name="c", subcore_axis_name="s")
    k = pl.kernel(
        _sc_body,
        out_type=jax.ShapeDtypeStruct((total, D), jnp.float32),
        mesh=mesh,
        scratch_types=[
            pltpu.VMEM((CHUNK,), jnp.int32),
            pltpu.VMEM((CHUNK,), jnp.int32),
            pltpu.VMEM((CHUNK, D), jnp.float32),
            pltpu.VMEM((CHUNK, D), jnp.float32),
            pltpu.SemaphoreType.DMA,
            pltpu.SemaphoreType.DMA,
        ],
        compiler_params=pltpu.CompilerParams(needs_layout_passes=False,
                                             use_tc_tiling_on_sc=False),
    )
    out = k(embeddings, idx1d)
    return out.reshape(B, L, D)
